# baseline (speedup 1.0000x reference)
import functools

import jax
import jax.numpy as jnp
from jax.experimental import pallas as pl
from jax.experimental.pallas import tpu as pltpu


def _y_kernel(feats_ref, nw_ref, ew_ref, y_ref):
    f = feats_ref[...]
    in_dim = f.shape[1]
    out = y_ref.shape[1] // 3
    y1 = jnp.dot(f, nw_ref[:in_dim, :], preferred_element_type=jnp.float32)
    y2 = jnp.dot(f, nw_ref[in_dim:, :], preferred_element_type=jnp.float32)
    y3 = jnp.dot(f, ew_ref[...], preferred_element_type=jnp.float32)
    y_ref[:, :out] = y1.astype(jnp.bfloat16)
    y_ref[:, out:2 * out] = y2.astype(jnp.bfloat16)
    y_ref[:, 2 * out:] = y3.astype(jnp.bfloat16)


def _main_kernel(nadj_ref, eadj_ref, y_ref, nb_ref, eb_ref, o_ref, acc_ref):
    i = pl.program_id(0)
    k = pl.program_id(1)
    nk = pl.num_programs(1)
    out = o_ref.shape[1]

    @pl.when(k == 0)
    def _zero():
        acc_ref[...] = jnp.zeros_like(acc_ref)

    a_n = nadj_ref[...]
    a_e = eadj_ref[...]
    k0 = k * a_n.shape[1]
    pos = (a_n > 0).astype(jnp.bfloat16)
    neg = (a_n < 0).astype(jnp.bfloat16)
    y = y_ref[pl.ds(k0, a_n.shape[1]), :]
    acc = jnp.dot(pos, y[:, :out], preferred_element_type=jnp.float32)
    acc += jnp.dot(neg, y[:, out:2 * out], preferred_element_type=jnp.float32)
    acc += jnp.dot(a_e.astype(jnp.bfloat16), y[:, 2 * out:], preferred_element_type=jnp.float32)

    bm, bk = a_n.shape
    @pl.when(jnp.logical_and(i * bm < (k + 1) * bk, k * bk < (i + 1) * bm))
    def _diag():
        col_off = pl.multiple_of(jnp.maximum(i * bm - k * bk, 0), bm)
        m = (jax.lax.broadcasted_iota(jnp.int32, (bm, bm), 0)
             == jax.lax.broadcasted_iota(jnp.int32, (bm, bm), 1))
        sub_e = eadj_ref[:, pl.ds(col_off, bm)]
        sub_n = nadj_ref[:, pl.ds(col_off, bm)]
        diag_e = jnp.sum(jnp.where(m, sub_e, 0.0), axis=1, keepdims=True)
        diag_n = jnp.sum(jnp.where(m, sub_n, 0.0), axis=1, keepdims=True)
        acc_ref[...] += diag_e * nb_ref[...] + diag_n * eb_ref[...]

    acc_ref[...] += acc

    @pl.when(k == nk - 1)
    def _flush():
        o_ref[...] = acc_ref[...]


@jax.jit
def kernel(feats, node_adj, edge_adj, node_weight, node_bias, edge_weight, edge_bias):
    n, in_dim = feats.shape
    out = node_bias.shape[0]

    y = pl.pallas_call(
        _y_kernel,
        out_shape=jax.ShapeDtypeStruct((n, 3 * out), jnp.bfloat16),
    )(feats, node_weight, edge_weight)

    bm = 256
    bk = 4096
    grid = (n // bm, n // bk)

    result = pl.pallas_call(
        _main_kernel,
        grid=grid,
        in_specs=[
            pl.BlockSpec((bm, bk), lambda i, k: (i, k)),
            pl.BlockSpec((bm, bk), lambda i, k: (i, k)),
            pl.BlockSpec((n, 3 * out), lambda i, k: (0, 0)),
            pl.BlockSpec((1, out), lambda i, k: (0, 0)),
            pl.BlockSpec((1, out), lambda i, k: (0, 0)),
        ],
        out_specs=pl.BlockSpec((bm, out), lambda i, k: (i, 0)),
        out_shape=jax.ShapeDtypeStruct((n, out), jnp.float32),
        scratch_shapes=[pltpu.VMEM((bm, out), jnp.float32)],
        compiler_params=pltpu.CompilerParams(
            dimension_semantics=("parallel", "arbitrary"),
        ),
    )(node_adj, edge_adj, y, node_bias.reshape(1, out), edge_bias.reshape(1, out))
    return result


# speedup vs baseline: 1.1282x; 1.1282x over previous
import functools

import jax
import jax.numpy as jnp
from jax.experimental import pallas as pl
from jax.experimental.pallas import tpu as pltpu


def _main_kernel(nadj_ref, eadj_ref, feats_ref, nw_ref, ew_ref, nb_ref, eb_ref,
                 o_ref, acc_ref, y_ref):
    i = pl.program_id(0)
    k = pl.program_id(1)
    nk = pl.num_programs(1)
    out = o_ref.shape[1]
    bm, bk = nadj_ref.shape

    @pl.when(jnp.logical_and(i == 0, k == 0))
    def _build_y():
        f = feats_ref[...]
        in_dim = f.shape[1]
        y1 = jnp.dot(f, nw_ref[:in_dim, :], preferred_element_type=jnp.float32)
        y2 = jnp.dot(f, nw_ref[in_dim:, :], preferred_element_type=jnp.float32)
        y3 = jnp.dot(f, ew_ref[...], preferred_element_type=jnp.float32)
        y_ref[:, :out] = y1.astype(jnp.bfloat16)
        y_ref[:, out:2 * out] = y2.astype(jnp.bfloat16)
        y_ref[:, 2 * out:] = y3.astype(jnp.bfloat16)

    @pl.when(k == 0)
    def _zero():
        acc_ref[...] = jnp.zeros_like(acc_ref)

    a_n = nadj_ref[...]
    a_e = eadj_ref[...]
    pos = (a_n > 0).astype(jnp.bfloat16)
    neg = (a_n < 0).astype(jnp.bfloat16)
    y = y_ref[pl.ds(k * bk, bk), :]
    acc = jnp.dot(pos, y[:, :out], preferred_element_type=jnp.float32)
    acc += jnp.dot(neg, y[:, out:2 * out], preferred_element_type=jnp.float32)
    acc += jnp.dot(a_e.astype(jnp.bfloat16), y[:, 2 * out:],
                   preferred_element_type=jnp.float32)

    @pl.when(jnp.logical_and(i * bm < (k + 1) * bk, k * bk < (i + 1) * bm))
    def _diag():
        col_off = pl.multiple_of(jnp.maximum(i * bm - k * bk, 0), bm)
        m = (jax.lax.broadcasted_iota(jnp.int32, (bm, bm), 0)
             == jax.lax.broadcasted_iota(jnp.int32, (bm, bm), 1))
        sub_e = eadj_ref[:, pl.ds(col_off, bm)]
        sub_n = nadj_ref[:, pl.ds(col_off, bm)]
        diag_e = jnp.sum(jnp.where(m, sub_e, 0.0), axis=1, keepdims=True)
        diag_n = jnp.sum(jnp.where(m, sub_n, 0.0), axis=1, keepdims=True)
        acc_ref[...] += diag_e * nb_ref[...] + diag_n * eb_ref[...]

    acc_ref[...] += acc

    @pl.when(k == nk - 1)
    def _flush():
        o_ref[...] = acc_ref[...]


@jax.jit
def kernel(feats, node_adj, edge_adj, node_weight, node_bias, edge_weight, edge_bias):
    n, in_dim = feats.shape
    out = node_bias.shape[0]

    bm = 512
    bk = 4096
    grid = (n // bm, n // bk)

    result = pl.pallas_call(
        _main_kernel,
        grid=grid,
        in_specs=[
            pl.BlockSpec((bm, bk), lambda i, k: (i, k)),
            pl.BlockSpec((bm, bk), lambda i, k: (i, k)),
            pl.BlockSpec((n, in_dim), lambda i, k: (0, 0)),
            pl.BlockSpec((2 * in_dim, out), lambda i, k: (0, 0)),
            pl.BlockSpec((in_dim, out), lambda i, k: (0, 0)),
            pl.BlockSpec((1, out), lambda i, k: (0, 0)),
            pl.BlockSpec((1, out), lambda i, k: (0, 0)),
        ],
        out_specs=pl.BlockSpec((bm, out), lambda i, k: (i, 0)),
        out_shape=jax.ShapeDtypeStruct((n, out), jnp.float32),
        scratch_shapes=[
            pltpu.VMEM((bm, out), jnp.float32),
            pltpu.VMEM((n, 3 * out), jnp.bfloat16),
        ],
        compiler_params=pltpu.CompilerParams(
            dimension_semantics=("arbitrary", "arbitrary"),
        ),
    )(node_adj, edge_adj, feats, node_weight,
      edge_weight, node_bias.reshape(1, out), edge_bias.reshape(1, out))
    return result
